# baseline (speedup 1.0000x reference)
import functools

import jax
import jax.numpy as jnp
from jax import lax
from jax.experimental import pallas as pl
from jax.experimental.pallas import tpu as pltpu
from jax.experimental.pallas import tpu_sc as plsc

E = 16
TOPK = 2
B = 128
D = 1024
F = 1024
NA = B * TOPK
LANES = 16
NCH = NA // LANES


def _sc_counts_body(ids_hbm, cnt_hbm, idvp, cntv):
    lane = lax.iota(jnp.int32, LANES)
    one = jnp.zeros((LANES,), jnp.int32) + 1
    neg = jnp.zeros((LANES,), jnp.int32) - 1
    idvp[pl.ds(0, LANES)] = neg
    idvp[pl.ds(LANES + NA, LANES)] = neg
    pltpu.sync_copy(ids_hbm, idvp.at[pl.ds(LANES, NA)])
    acc = jnp.zeros((LANES,), jnp.int32)
    for t in range(1, LANES + NA):
        v = idvp[pl.ds(t, LANES)]
        acc = acc + (one - jnp.minimum(one, jnp.abs(v - lane)))
    cntv[...] = acc
    pltpu.sync_copy(cntv, cnt_hbm)


def _sc_counts(ids_flat):
    mesh = plsc.VectorSubcoreMesh(core_axis_name="c", subcore_axis_name="s")
    return functools.partial(
        pl.kernel,
        mesh=mesh,
        out_type=jax.ShapeDtypeStruct((E,), jnp.int32),
        scratch_types=[
            pltpu.VMEM((NA + 2 * LANES,), jnp.int32),
            pltpu.VMEM((LANES,), jnp.int32),
        ],
    )(_sc_counts_body)(ids_flat)


def _moe_body(x_ref, ids_ref, es_ref, w1g_ref, w1u_ref, w1sg_ref, w1su_ref,
              w2_ref, w2s_ref, out_ref):
    e = pl.program_id(0)

    xv = x_ref[...].astype(jnp.bfloat16)
    w1g = w1g_ref[0].astype(jnp.bfloat16)
    w1u = w1u_ref[0].astype(jnp.bfloat16)
    gate = jnp.dot(xv, w1g, preferred_element_type=jnp.float32) * w1sg_ref[0]
    up = jnp.dot(xv, w1u, preferred_element_type=jnp.float32) * w1su_ref[0]
    h = gate * jax.nn.sigmoid(gate) * up
    y2 = jnp.dot(h.astype(jnp.bfloat16), w2_ref[0].astype(jnp.bfloat16),
                 preferred_element_type=jnp.float32) * w2s_ref[0]

    m = ids_ref[...] == e
    w = jnp.sum(jnp.where(m, es_ref[...], 0.0), axis=1, keepdims=True)
    contrib = w * y2

    @pl.when(e == 0)
    def _():
        out_ref[...] = contrib

    @pl.when(e != 0)
    def _():
        out_ref[...] += contrib


@jax.jit
def kernel(x, expert_ids, smooth_scales, expert_scales, x_active_mask,
           gmm1_weight, gmm1_weight_scale, gmm2_weight, gmm2_weight_scale):
    del smooth_scales, x_active_mask
    w1s3 = gmm1_weight_scale.reshape(E, 1, 2 * F)
    w2s3 = gmm2_weight_scale.reshape(E, 1, D)

    counts = _sc_counts(expert_ids.reshape(NA))

    out = pl.pallas_call(
        _moe_body,
        grid=(E,),
        in_specs=[
            pl.BlockSpec((B, D), lambda e: (0, 0)),
            pl.BlockSpec((B, TOPK), lambda e: (0, 0)),
            pl.BlockSpec((B, TOPK), lambda e: (0, 0)),
            pl.BlockSpec((1, D, F), lambda e: (e, 0, 0)),
            pl.BlockSpec((1, D, F), lambda e: (e, 0, 1)),
            pl.BlockSpec((1, 1, F), lambda e: (e, 0, 0)),
            pl.BlockSpec((1, 1, F), lambda e: (e, 0, 1)),
            pl.BlockSpec((1, F, D), lambda e: (e, 0, 0)),
            pl.BlockSpec((1, 1, D), lambda e: (e, 0, 0)),
        ],
        out_specs=pl.BlockSpec((B, D), lambda e: (0, 0)),
        out_shape=jax.ShapeDtypeStruct((B, D), jnp.float32),
        compiler_params=pltpu.CompilerParams(
            dimension_semantics=("arbitrary",),
        ),
    )(x, expert_ids, expert_scales, gmm1_weight, gmm1_weight,
      w1s3, w1s3, gmm2_weight, w2s3)
    return out, counts


# speedup vs baseline: 1.2238x; 1.2238x over previous
import functools

import jax
import jax.numpy as jnp
from jax.experimental import pallas as pl
from jax.experimental.pallas import tpu as pltpu

E = 16
TOPK = 2
B = 128
D = 1024
F = 1024
NF = 1
FB = F // NF


def _moe_body(x_ref, ids_ref, es_ref, w1g_ref, w1u_ref, w1sg_ref, w1su_ref,
              w2_ref, w2s_ref, out_ref, cnt_ref):
    e = pl.program_id(0)
    f = pl.program_id(1)

    xv = x_ref[...].astype(jnp.bfloat16)
    w1g = w1g_ref[0].astype(jnp.bfloat16)
    w1u = w1u_ref[0].astype(jnp.bfloat16)
    gate = jnp.dot(xv, w1g, preferred_element_type=jnp.float32) * w1sg_ref[0]
    up = jnp.dot(xv, w1u, preferred_element_type=jnp.float32) * w1su_ref[0]
    h = gate * jax.nn.sigmoid(gate) * up
    y2 = jnp.dot(h.astype(jnp.bfloat16), w2_ref[0].astype(jnp.bfloat16),
                 preferred_element_type=jnp.float32) * w2s_ref[0]

    m = ids_ref[...] == e
    w = jnp.sum(jnp.where(m, es_ref[...], 0.0), axis=1, keepdims=True)
    contrib = w * y2

    first = (e == 0) & (f == 0)

    @pl.when(first)
    def _():
        out_ref[...] = contrib

    @pl.when(jnp.logical_not(first))
    def _():
        out_ref[...] += contrib

    @pl.when(f == 0)
    def _():
        cnt_ref[e] = jnp.sum(m.astype(jnp.int32))


@jax.jit
def kernel(x, expert_ids, smooth_scales, expert_scales, x_active_mask,
           gmm1_weight, gmm1_weight_scale, gmm2_weight, gmm2_weight_scale):
    del smooth_scales, x_active_mask
    w1s3 = gmm1_weight_scale.reshape(E, 1, 2 * F)
    w2s3 = gmm2_weight_scale.reshape(E, 1, D)

    out, counts = pl.pallas_call(
        _moe_body,
        grid=(E, NF),
        in_specs=[
            pl.BlockSpec((B, D), lambda e, f: (0, 0)),
            pl.BlockSpec((B, TOPK), lambda e, f: (0, 0)),
            pl.BlockSpec((B, TOPK), lambda e, f: (0, 0)),
            pl.BlockSpec((1, D, FB), lambda e, f: (e, 0, f)),
            pl.BlockSpec((1, D, FB), lambda e, f: (e, 0, f + NF)),
            pl.BlockSpec((1, 1, FB), lambda e, f: (e, 0, f)),
            pl.BlockSpec((1, 1, FB), lambda e, f: (e, 0, f + NF)),
            pl.BlockSpec((1, FB, D), lambda e, f: (e, f, 0)),
            pl.BlockSpec((1, 1, D), lambda e, f: (e, 0, 0)),
        ],
        out_specs=[
            pl.BlockSpec((B, D), lambda e, f: (0, 0)),
            pl.BlockSpec(memory_space=pltpu.SMEM),
        ],
        out_shape=[
            jax.ShapeDtypeStruct((B, D), jnp.float32),
            jax.ShapeDtypeStruct((E,), jnp.int32),
        ],
        compiler_params=pltpu.CompilerParams(
            dimension_semantics=("arbitrary", "arbitrary"),
        ),
    )(x, expert_ids, expert_scales, gmm1_weight, gmm1_weight,
      w1s3, w1s3, gmm2_weight, w2s3)
    return out, counts
